# baseline (speedup 1.0000x reference)
import functools

import jax
import jax.numpy as jnp
from jax import lax
from jax.experimental import pallas as pl
from jax.experimental.pallas import tpu as pltpu
from jax.experimental.pallas import tpu_sc as plsc

N_FTS = 100000
D = 256
B = 16384
L = 32

NC = 2
NS = 16
NW = NC * NS
LANES = 16

SAMPLES = 2 * B
SPW = SAMPLES // NW
CHUNK = 4
NBUF = 3
NCH = SPW // CHUNK


def _ft_body(ics_hbm, vals_hbm, table_hbm, out_hbm, icsr_v, valsr_v, idxl_v,
             rows_v, accs_v, isems, sems, osems):
    wid = lax.axis_index("s") * NC + lax.axis_index("c")
    base = wid * SPW
    RL = CHUNK * L

    def start_idx(g, b):
        pltpu.async_copy(
            ics_hbm.at[pl.ds(base + g * CHUNK, CHUNK), :],
            icsr_v.at[pl.ds(b * CHUNK, CHUNK), :],
            isems[b],
        )
        pltpu.async_copy(
            vals_hbm.at[pl.ds(base + g * CHUNK, CHUNK), :],
            valsr_v.at[pl.ds(b * CHUNK, CHUNK), :],
            isems[b],
        )

    def wait_idx(b):
        pltpu.make_async_copy(
            ics_hbm.at[pl.ds(0, CHUNK), :],
            icsr_v.at[pl.ds(b * CHUNK, CHUNK), :],
            isems[b],
        ).wait()
        pltpu.make_async_copy(
            vals_hbm.at[pl.ds(0, CHUNK), :],
            valsr_v.at[pl.ds(b * CHUNK, CHUNK), :],
            isems[b],
        ).wait()

    def start_gather(b):
        for i in range(CHUNK):
            row = b * CHUNK + i
            idxl_v[pl.ds(b * RL + i * L, LANES)] = icsr_v[row, pl.ds(0, LANES)]
            idxl_v[pl.ds(b * RL + i * L + LANES, LANES)] = icsr_v[row, pl.ds(LANES, LANES)]
        pltpu.async_copy(
            table_hbm.at[idxl_v.at[pl.ds(b * RL, RL)]],
            rows_v.at[pl.ds(b * RL, RL)],
            sems[b],
        )

    def wait_gather(b):
        pltpu.make_async_copy(
            table_hbm.at[pl.ds(0, RL)], rows_v.at[pl.ds(b * RL, RL)], sems[b]
        ).wait()

    def compute_chunk(g, b):
        def sample_body(i, carry2):
            row = b * CHUNK + i
            v0 = valsr_v[row, pl.ds(0, LANES)]
            v1 = valsr_v[row, pl.ds(LANES, LANES)]
            rbase = b * RL + i * L

            def j_body(j, carry3):
                col = pl.multiple_of(j * LANES, LANES)
                part = [jnp.zeros((LANES,), jnp.float32) for _ in range(4)]
                for l in range(L):
                    vv = v0 if l < LANES else v1
                    val = lax.index_in_dim(vv, l % LANES, 0, keepdims=False)
                    part[l % 4] = part[l % 4] + rows_v[rbase + l, pl.ds(col, LANES)] * val
                acc = (part[0] + part[1]) + (part[2] + part[3])
                accs_v[b * CHUNK + i, pl.ds(col, LANES)] = acc
                return carry3

            lax.fori_loop(0, D // LANES, j_body, 0)
            return carry2

        lax.fori_loop(0, CHUNK, sample_body, 0)
        pltpu.async_copy(
            accs_v.at[pl.ds(b * CHUNK, CHUNK)],
            out_hbm.at[pl.ds(base + g * CHUNK, CHUNK)],
            osems[b],
        )

    def wait_out(b):
        pltpu.make_async_copy(
            accs_v.at[pl.ds(b * CHUNK, CHUNK)],
            out_hbm.at[pl.ds(0, CHUNK)],
            osems[b],
        ).wait()

    for x in range(NBUF):
        start_idx(x, x)
    for x in range(NBUF - 1):
        wait_idx(x)
        start_gather(x)

    def step(g, b, first_round):
        wait_gather(b)
        nxt = g + NBUF - 1

        @pl.when(nxt < NCH)
        def _():
            nb = (b + NBUF - 1) % NBUF
            wait_idx(nb)
            start_gather(nb)

        @pl.when(jnp.logical_not(first_round))
        def _():
            wait_out(b)

        compute_chunk(g, b)

        @pl.when(g + NBUF < NCH)
        def _():
            start_idx(g + NBUF, b)

    def ring_body(q, carry2):
        for b in range(NBUF):
            step(NBUF * q + b, b, q < 1)
        return carry2

    n_full = NCH // NBUF
    lax.fori_loop(0, n_full, ring_body, 0)
    for b in range(NCH - n_full * NBUF):
        step(jnp.int32(n_full * NBUF + b), b, jnp.bool_(False))
    for b in range(NBUF):
        wait_out(b)


def _feature_transform(ics2, vals2, table):
    mesh = plsc.VectorSubcoreMesh(core_axis_name="c", subcore_axis_name="s")
    return pl.kernel(
        _ft_body,
        out_type=jax.ShapeDtypeStruct((SAMPLES, D), jnp.float32),
        mesh=mesh,
        scratch_types=[
            pltpu.VMEM((NBUF * CHUNK, L), jnp.int32),
            pltpu.VMEM((NBUF * CHUNK, L), jnp.float32),
            pltpu.VMEM((NBUF * CHUNK * L,), jnp.int32),
            pltpu.VMEM((NBUF * CHUNK * L, D), jnp.float32),
            pltpu.VMEM((NBUF * CHUNK, D), jnp.float32),
            [pltpu.SemaphoreType.DMA for _ in range(NBUF)],
            [pltpu.SemaphoreType.DMA for _ in range(NBUF)],
            [pltpu.SemaphoreType.DMA for _ in range(NBUF)],
        ],
        name="nnue_feature_transform",
    )(ics2, vals2, table)


def _mlp_body(wf_ref, bf_ref, s_ref, bft_ref, W1_ref, b1_ref, W2_ref, b2_ref,
              Wo_ref, bo_ref, o_ref):
    bft = bft_ref[...]
    wf = wf_ref[...] + bft
    bf = bf_ref[...] + bft
    s = s_ref[...]
    x1 = jnp.clip((1.0 - s) * wf + s * bf, 0.0, 1.0)
    x2 = jnp.clip((1.0 - s) * bf + s * wf, 0.0, 1.0)
    dn = (((1,), (1,)), ((), ()))
    h = lax.dot_general(x1, W1_ref[:, :D], dn, preferred_element_type=jnp.float32)
    h += lax.dot_general(x2, W1_ref[:, D:], dn, preferred_element_type=jnp.float32)
    h = jnp.clip(h + b1_ref[...], 0.0, 1.0)
    h = lax.dot_general(h, W2_ref[...], dn, preferred_element_type=jnp.float32)
    h = jnp.clip(h + b2_ref[...], 0.0, 1.0)
    o_ref[...] = jnp.sum(h * Wo_ref[...], axis=1, keepdims=True) + bo_ref[...]


def _mlp_head(fts, stm, b_ft, W1, b1, W2, b2, Wo, bo):
    BB = 2048
    grid = (B // BB,)
    return pl.pallas_call(
        _mlp_body,
        grid=grid,
        in_specs=[
            pl.BlockSpec((BB, D), lambda i: (i, 0)),
            pl.BlockSpec((BB, D), lambda i: (B // BB + i, 0)),
            pl.BlockSpec((BB, 1), lambda i: (i, 0)),
            pl.BlockSpec((1, D), lambda i: (0, 0)),
            pl.BlockSpec((32, 2 * D), lambda i: (0, 0)),
            pl.BlockSpec((1, 32), lambda i: (0, 0)),
            pl.BlockSpec((32, 32), lambda i: (0, 0)),
            pl.BlockSpec((1, 32), lambda i: (0, 0)),
            pl.BlockSpec((1, 32), lambda i: (0, 0)),
            pl.BlockSpec((1, 1), lambda i: (0, 0)),
        ],
        out_specs=pl.BlockSpec((BB, 1), lambda i: (i, 0)),
        out_shape=jax.ShapeDtypeStruct((B, 1), jnp.float32),
    )(fts, fts, stm, b_ft, W1, b1, W2, b2, Wo, bo)


def kernel(wft_ics, wft_vals, bft_ics, bft_vals, stm, W_ft, b_ft, W1, b1, W2, b2, Wo, bo):
    ics2 = jnp.concatenate([wft_ics, bft_ics], axis=0)
    vals2 = jnp.concatenate([wft_vals, bft_vals], axis=0)
    fts = _feature_transform(ics2, vals2, W_ft)
    return _mlp_head(
        fts, stm,
        b_ft.reshape(1, D),
        W1, b1.reshape(1, 32), W2, b2.reshape(1, 32),
        Wo.reshape(1, 32), bo.reshape(1, 1),
    )


# speedup vs baseline: 1.2962x; 1.2962x over previous
import functools

import jax
import jax.numpy as jnp
from jax import lax
from jax.experimental import pallas as pl
from jax.experimental.pallas import tpu as pltpu
from jax.experimental.pallas import tpu_sc as plsc

N_FTS = 100000
D = 256
B = 16384
L = 32

NC = 2
NS = 16
NW = NC * NS
LANES = 16

SAMPLES = 2 * B
SPW = SAMPLES // NW
CHUNK = 4
NBUF = 3
KC = 8
NST = SPW // (KC * CHUNK)
NCH = SPW // CHUNK


def _ft_body(ics_hbm, vals_hbm, table_hbm, out_hbm, icsr_v, valsr_v, idxl_v,
             rows_v, accs_v, isems, sems, osems):
    wid = lax.axis_index("s") * NC + lax.axis_index("c")
    base = wid * SPW
    RL = CHUNK * L
    KS = KC * CHUNK

    def start_stage(st, p):
        pltpu.async_copy(
            ics_hbm.at[pl.ds(base + st * KS, KS), :],
            icsr_v.at[pl.ds(p * KS, KS), :],
            isems[p],
        )
        pltpu.async_copy(
            vals_hbm.at[pl.ds(base + st * KS, KS), :],
            valsr_v.at[pl.ds(p * KS, KS), :],
            isems[p],
        )

    def wait_stage(p):
        pltpu.make_async_copy(
            ics_hbm.at[pl.ds(0, KS), :],
            icsr_v.at[pl.ds(p * KS, KS), :],
            isems[p],
        ).wait()
        pltpu.make_async_copy(
            vals_hbm.at[pl.ds(0, KS), :],
            valsr_v.at[pl.ds(p * KS, KS), :],
            isems[p],
        ).wait()

    def start_gather(g, b, p):
        for i in range(CHUNK):
            row = p * KS + (g % KC) * CHUNK + i
            idxl_v[pl.ds(b * RL + i * L, LANES)] = icsr_v[row, pl.ds(0, LANES)]
            idxl_v[pl.ds(b * RL + i * L + LANES, LANES)] = icsr_v[row, pl.ds(LANES, LANES)]
        pltpu.async_copy(
            table_hbm.at[idxl_v.at[pl.ds(b * RL, RL)]],
            rows_v.at[pl.ds(b * RL, RL)],
            sems[b],
        )

    def start_gather_psel(g, b):
        p = (g // KC) % 2

        @pl.when(p == 0)
        def _():
            start_gather(g, b, 0)

        @pl.when(p == 1)
        def _():
            start_gather(g, b, 1)

    def wait_gather(b):
        pltpu.make_async_copy(
            table_hbm.at[pl.ds(0, RL)], rows_v.at[pl.ds(b * RL, RL)], sems[b]
        ).wait()

    def compute_chunk(g, b):
        p = (g // KC) % 2

        def sample_body(i, carry2):
            row = p * KS + (g % KC) * CHUNK + i
            v0 = valsr_v[row, pl.ds(0, LANES)]
            v1 = valsr_v[row, pl.ds(LANES, LANES)]
            rbase = b * RL + i * L

            def j_body(j, carry3):
                col = pl.multiple_of(j * LANES, LANES)
                part = [jnp.zeros((LANES,), jnp.float32) for _ in range(4)]
                for l in range(L):
                    vv = v0 if l < LANES else v1
                    val = lax.index_in_dim(vv, l % LANES, 0, keepdims=False)
                    part[l % 4] = part[l % 4] + rows_v[rbase + l, pl.ds(col, LANES)] * val
                acc = (part[0] + part[1]) + (part[2] + part[3])
                accs_v[b * CHUNK + i, pl.ds(col, LANES)] = acc
                return carry3

            lax.fori_loop(0, D // LANES, j_body, 0)
            return carry2

        lax.fori_loop(0, CHUNK, sample_body, 0)
        pltpu.async_copy(
            accs_v.at[pl.ds(b * CHUNK, CHUNK)],
            out_hbm.at[pl.ds(base + g * CHUNK, CHUNK)],
            osems[b],
        )

    def wait_out(b):
        pltpu.make_async_copy(
            accs_v.at[pl.ds(b * CHUNK, CHUNK)],
            out_hbm.at[pl.ds(0, CHUNK)],
            osems[b],
        ).wait()

    start_stage(0, 0)
    start_stage(1, 1)
    wait_stage(0)
    for x in range(NBUF - 1):
        start_gather(x, x, 0)

    def step(g, b, first_round):
        wait_gather(b)
        nxt = g + NBUF - 1

        @pl.when(jnp.logical_and(nxt < NCH, nxt % KC == 0))
        def _():
            p_in = (nxt // KC) % 2

            @pl.when(p_in == 0)
            def _():
                wait_stage(0)

            @pl.when(p_in == 1)
            def _():
                wait_stage(1)

        @pl.when(nxt < NCH)
        def _():
            start_gather_psel(nxt, (b + NBUF - 1) % NBUF)

        @pl.when(jnp.logical_not(first_round))
        def _():
            wait_out(b)

        compute_chunk(g, b)

        @pl.when(g % KC == KC - 1)
        def _():
            st_next = g // KC + 2

            @pl.when(st_next < NST)
            def _():
                p_next = st_next % 2

                @pl.when(p_next == 0)
                def _():
                    start_stage(st_next, 0)

                @pl.when(p_next == 1)
                def _():
                    start_stage(st_next, 1)

    def ring_body(q, carry2):
        for b in range(NBUF):
            step(NBUF * q + b, b, q < 1)
        return carry2

    n_full = NCH // NBUF
    lax.fori_loop(0, n_full, ring_body, 0)
    for b in range(NCH - n_full * NBUF):
        step(jnp.int32(n_full * NBUF + b), b, jnp.bool_(False))
    for b in range(NBUF):
        wait_out(b)


def _feature_transform(ics2, vals2, table):
    mesh = plsc.VectorSubcoreMesh(core_axis_name="c", subcore_axis_name="s")
    return pl.kernel(
        _ft_body,
        out_type=jax.ShapeDtypeStruct((SAMPLES, D), jnp.float32),
        mesh=mesh,
        scratch_types=[
            pltpu.VMEM((2 * KC * CHUNK, L), jnp.int32),
            pltpu.VMEM((2 * KC * CHUNK, L), jnp.float32),
            pltpu.VMEM((NBUF * CHUNK * L,), jnp.int32),
            pltpu.VMEM((NBUF * CHUNK * L, D), jnp.float32),
            pltpu.VMEM((NBUF * CHUNK, D), jnp.float32),
            [pltpu.SemaphoreType.DMA for _ in range(2)],
            [pltpu.SemaphoreType.DMA for _ in range(NBUF)],
            [pltpu.SemaphoreType.DMA for _ in range(NBUF)],
        ],
        name="nnue_feature_transform",
    )(ics2, vals2, table)


def _mlp_body(wf_ref, bf_ref, s_ref, bft_ref, W1_ref, b1_ref, W2_ref, b2_ref,
              Wo_ref, bo_ref, o_ref):
    bft = bft_ref[...]
    wf = wf_ref[...] + bft
    bf = bf_ref[...] + bft
    s = s_ref[...]
    x1 = jnp.clip((1.0 - s) * wf + s * bf, 0.0, 1.0)
    x2 = jnp.clip((1.0 - s) * bf + s * wf, 0.0, 1.0)
    dn = (((1,), (1,)), ((), ()))
    h = lax.dot_general(x1, W1_ref[:, :D], dn, preferred_element_type=jnp.float32)
    h += lax.dot_general(x2, W1_ref[:, D:], dn, preferred_element_type=jnp.float32)
    h = jnp.clip(h + b1_ref[...], 0.0, 1.0)
    h = lax.dot_general(h, W2_ref[...], dn, preferred_element_type=jnp.float32)
    h = jnp.clip(h + b2_ref[...], 0.0, 1.0)
    o_ref[...] = jnp.sum(h * Wo_ref[...], axis=1, keepdims=True) + bo_ref[...]


def _mlp_head(fts, stm, b_ft, W1, b1, W2, b2, Wo, bo):
    BB = 2048
    grid = (B // BB,)
    return pl.pallas_call(
        _mlp_body,
        grid=grid,
        in_specs=[
            pl.BlockSpec((BB, D), lambda i: (i, 0)),
            pl.BlockSpec((BB, D), lambda i: (B // BB + i, 0)),
            pl.BlockSpec((BB, 1), lambda i: (i, 0)),
            pl.BlockSpec((1, D), lambda i: (0, 0)),
            pl.BlockSpec((32, 2 * D), lambda i: (0, 0)),
            pl.BlockSpec((1, 32), lambda i: (0, 0)),
            pl.BlockSpec((32, 32), lambda i: (0, 0)),
            pl.BlockSpec((1, 32), lambda i: (0, 0)),
            pl.BlockSpec((1, 32), lambda i: (0, 0)),
            pl.BlockSpec((1, 1), lambda i: (0, 0)),
        ],
        out_specs=pl.BlockSpec((BB, 1), lambda i: (i, 0)),
        out_shape=jax.ShapeDtypeStruct((B, 1), jnp.float32),
    )(fts, fts, stm, b_ft, W1, b1, W2, b2, Wo, bo)


def kernel(wft_ics, wft_vals, bft_ics, bft_vals, stm, W_ft, b_ft, W1, b1, W2, b2, Wo, bo):
    ics2 = jnp.concatenate([wft_ics, bft_ics], axis=0)
    vals2 = jnp.concatenate([wft_vals, bft_vals], axis=0)
    fts = _feature_transform(ics2, vals2, W_ft)
    return _mlp_head(
        fts, stm,
        b_ft.reshape(1, D),
        W1, b1.reshape(1, 32), W2, b2.reshape(1, 32),
        Wo.reshape(1, 32), bo.reshape(1, 1),
    )
